# baseline (speedup 1.0000x reference)
import dataclasses
import functools

import jax
import jax.numpy as jnp
from jax.experimental import pallas as pl
from jax.experimental.pallas import tpu as pltpu
from jax.experimental.pallas import tpu_sc as plsc

N = 1048576
DIM = 64
BINS = (1, 2, 3, 4, 8, 16, 32, 64)
CH = 128
LANES = 16


def kernel(lengths, table):
    lengths = lengths.astype(jnp.int32).reshape(1, N)
    mesh = plsc.VectorSubcoreMesh(core_axis_name="c", subcore_axis_name="s")
    cp = pltpu.CompilerParams()
    if "needs_layout_passes" in pltpu.CompilerParams.__dataclass_fields__:
        cp = dataclasses.replace(cp, needs_layout_passes=False)
    cp = dataclasses.replace(cp, use_tc_tiling_on_sc=False)

    @functools.partial(
        pl.kernel,
        out_type=jax.ShapeDtypeStruct((N, DIM), jnp.float32),
        mesh=mesh,
        scratch_types=[pltpu.VMEM((1, CH), jnp.int32)],
        compiler_params=cp,
    )
    def k(len_hbm, tab_hbm, out_hbm, idx_v):
        def body(len_vmem, out_vmem):
            @pl.loop(0, CH, step=LANES)
            def _(c):
                v = len_vmem[0, pl.ds(c, LANES)]
                acc = (v >= BINS[0]).astype(jnp.int32)
                for b in BINS[1:]:
                    acc += (v >= b).astype(jnp.int32)
                idx_v[0, pl.ds(c, LANES)] = acc

            pltpu.sync_copy(tab_hbm.at[idx_v.at[0]], out_vmem)

        pltpu.emit_pipeline(
            body,
            grid=(N // CH,),
            in_specs=[pl.BlockSpec((1, CH), lambda i: (0, i))],
            out_specs=[pl.BlockSpec((CH, DIM), lambda i: (i, 0))],
            core_axis_name=("c", "s"),
            dimension_semantics=(pltpu.PARALLEL,),
        )(len_hbm, out_hbm)

    return k(lengths, table)


# speedup vs baseline: 13.0824x; 13.0824x over previous
import dataclasses
import functools

import jax
import jax.numpy as jnp
from jax.experimental import pallas as pl
from jax.experimental.pallas import tpu as pltpu
from jax.experimental.pallas import tpu_sc as plsc

N = 1048576
DIM = 64
BINS = (1, 2, 3, 4, 8, 16, 32, 64)
CH = 128
LANES = 16


def kernel(lengths, table):
    lengths = lengths.astype(jnp.int32).reshape(1, N)
    mesh = plsc.VectorSubcoreMesh(core_axis_name="c", subcore_axis_name="s")
    cp = pltpu.CompilerParams()
    if "needs_layout_passes" in pltpu.CompilerParams.__dataclass_fields__:
        cp = dataclasses.replace(cp, needs_layout_passes=False)
    cp = dataclasses.replace(cp, use_tc_tiling_on_sc=False)

    @functools.partial(
        pl.kernel,
        out_type=jax.ShapeDtypeStruct((N, DIM), jnp.float32),
        mesh=mesh,
        scratch_types=[
            pltpu.VMEM((1, CH), jnp.int32),
            pltpu.VMEM_SHARED((9, DIM), jnp.float32),
        ],
        compiler_params=cp,
    )
    def k(len_hbm, tab_hbm, out_hbm, idx_v, tab_v):
        pltpu.sync_copy(tab_hbm, tab_v)

        def body(len_vmem, out_vmem):
            @pl.loop(0, CH, step=LANES)
            def _(c):
                v = len_vmem[0, pl.ds(c, LANES)]
                acc = (v >= BINS[0]).astype(jnp.int32)
                for b in BINS[1:]:
                    acc += (v >= b).astype(jnp.int32)
                idx_v[0, pl.ds(c, LANES)] = acc

            pltpu.sync_copy(tab_v.at[idx_v.at[0]], out_vmem)

        pltpu.emit_pipeline(
            body,
            grid=(N // CH,),
            in_specs=[pl.BlockSpec((1, CH), lambda i: (0, i))],
            out_specs=[pl.BlockSpec((CH, DIM), lambda i: (i, 0))],
            core_axis_name=("c", "s"),
            dimension_semantics=(pltpu.PARALLEL,),
        )(len_hbm, out_hbm)

    return k(lengths, table)
